# baseline (speedup 1.0000x reference)
import jax
import jax.numpy as jnp
from jax import lax
from jax.experimental import pallas as pl
from jax.experimental.pallas import tpu as pltpu
from jax.experimental.pallas import tpu_sc as plsc

_NUM_HASH = 3
_NUM_WORDS = 1000000
_EMB = 64
_NC = 2
_NS = 16
_NW = _NC * _NS
_LANES = 16

_N = 4096 * 50
_NPW = _N // _NW
_C = 128
_NCHUNK = _NPW // _C
_NPAIR = _NCHUNK // 2


def _body(ids_hbm, hash0_hbm, hash1_hbm, hash2_hbm,
          imp0_hbm, imp1_hbm, imp2_hbm, emb_hbm, out_hbm,
          ids_v,
          bidxa, iidxa, buck0a, buck1a, buck2a,
          wimp0a, wimp1a, wimp2a, emb0a, emb1a, emb2a, outba,
          bidxb, iidxb, buck0b, buck1b, buck2b,
          wimp0b, wimp1b, wimp2b, emb0b, emb1b, emb2b, outbb,
          sem_s, sem_e, sem_o):
    sets = (
        dict(bidx=bidxa, iidx=iidxa,
             buck=(buck0a, buck1a, buck2a), imp=(wimp0a, wimp1a, wimp2a),
             emb=(emb0a, emb1a, emb2a), outb=outba),
        dict(bidx=bidxb, iidx=iidxb,
             buck=(buck0b, buck1b, buck2b), imp=(wimp0b, wimp1b, wimp2b),
             emb=(emb0b, emb1b, emb2b), outb=outbb),
    )
    imp_hbms = (imp0_hbm, imp1_hbm, imp2_hbm)
    hash_hbms = (hash0_hbm, hash1_hbm, hash2_hbm)
    wid = lax.axis_index("s") * _NC + lax.axis_index("c")
    base = wid * _NPW
    pltpu.sync_copy(ids_hbm.at[pl.ds(base, _NPW)], ids_v)

    def build_idx(off, s):
        for j in range(_C // _LANES):
            dsl = pl.ds(j * _LANES, _LANES)
            v = ids_v[pl.ds(off + j * _LANES, _LANES)]
            s['bidx'][dsl] = v
            vi = v + 3
            s['iidx'][dsl] = jnp.where(vi >= _NUM_WORDS, vi - _NUM_WORDS, vi)

    def fire_w1(s):
        for h in range(_NUM_HASH):
            pltpu.async_copy(hash_hbms[h].at[s['bidx']], s['buck'][h], sem_s)
            pltpu.async_copy(imp_hbms[h].at[s['iidx']], s['imp'][h], sem_s)

    def wait_w1(s):
        for h in range(_NUM_HASH):
            pltpu.make_async_copy(hash_hbms[h].at[s['bidx']], s['buck'][h],
                                  sem_s).wait()
            pltpu.make_async_copy(imp_hbms[h].at[s['iidx']], s['imp'][h],
                                  sem_s).wait()

    def fire_w2(s):
        for h in range(_NUM_HASH):
            pltpu.async_copy(emb_hbm.at[s['buck'][h]], s['emb'][h], sem_e)

    def wait_w2(s):
        for h in range(_NUM_HASH):
            pltpu.make_async_copy(emb_hbm.at[s['buck'][h]], s['emb'][h],
                                  sem_e).wait()

    def combine(s):
        e0, e1, e2 = s['emb']
        i0, i1, i2 = s['imp']
        ob = s['outb']
        dnums = lax.GatherDimensionNumbers(
            offset_dims=(), collapsed_slice_dims=(0,), start_index_map=(0,))

        def bcast(vec, k):
            idx = jnp.full((_LANES, 1), k, jnp.int32)
            return lax.gather(vec, idx, dnums, (1,),
                              mode=lax.GatherScatterMode.PROMISE_IN_BOUNDS)

        def grp(g, c):
            t0 = g * _LANES
            m0 = i0[pl.ds(t0, _LANES)]
            m1 = i1[pl.ds(t0, _LANES)]
            m2 = i2[pl.ds(t0, _LANES)]
            for k in range(_LANES):
                t = t0 + k
                w0 = bcast(m0, k)
                w1 = bcast(m1, k)
                w2 = bcast(m2, k)
                for d in range(_EMB // _LANES):
                    esl = pl.ds(d * _LANES, _LANES)
                    ob[t, esl] = (e0[t, esl] * w0 + e1[t, esl] * w1
                                  + e2[t, esl] * w2)
            return c
        lax.fori_loop(0, _C // _LANES, grp, 0)

    def chunk_step(i, b):
        s = sets[b]
        nxt = sets[1 - b]
        j = 2 * i + b
        off = j * _C

        @pl.when(j >= 2)
        def _():
            pltpu.make_async_copy(
                s['outb'], out_hbm.at[pl.ds(base + off - 2 * _C, _C)],
                sem_o).wait()

        wait_w2(s)

        @pl.when(j + 2 < _NCHUNK)
        def _():
            build_idx(off + 2 * _C, s)

        @pl.when(j + 1 < _NCHUNK)
        def _():
            wait_w1(nxt)
            fire_w2(nxt)

        combine(s)
        pltpu.async_copy(s['outb'], out_hbm.at[pl.ds(base + off, _C)], sem_o)

        @pl.when(j + 2 < _NCHUNK)
        def _():
            fire_w1(s)

    def pair(i, carry):
        chunk_step(i, 0)
        chunk_step(i, 1)
        return carry

    build_idx(0, sets[0])
    fire_w1(sets[0])
    build_idx(_C, sets[1])
    fire_w1(sets[1])
    wait_w1(sets[0])
    fire_w2(sets[0])
    lax.fori_loop(0, _NPAIR, pair, 0)
    pltpu.make_async_copy(
        sets[0]['outb'], out_hbm.at[pl.ds(base + _NPW - 2 * _C, _C)],
        sem_o).wait()
    pltpu.make_async_copy(
        sets[1]['outb'], out_hbm.at[pl.ds(base + _NPW - _C, _C)],
        sem_o).wait()


def _sc_lookup(ids, hash_rows, imp_cols, emb):
    mesh = plsc.VectorSubcoreMesh(core_axis_name="c", subcore_axis_name="s")
    one_set = (
        [pltpu.VMEM((_C,), jnp.int32)] * 2
        + [pltpu.VMEM((_C,), jnp.int32)] * 3
        + [pltpu.VMEM((_C,), jnp.float32)] * 3
        + [pltpu.VMEM((_C, _EMB), jnp.float32)] * 3
        + [pltpu.VMEM((_C, _EMB), jnp.float32)]
    )
    f = pl.kernel(
        _body,
        out_type=jax.ShapeDtypeStruct((_N, _EMB), jnp.float32),
        mesh=mesh,
        compiler_params=pltpu.CompilerParams(needs_layout_passes=False,
                                             use_tc_tiling_on_sc=False),
        scratch_types=(
            [pltpu.VMEM((_NPW,), jnp.int32)]
            + one_set + one_set
            + [pltpu.SemaphoreType.DMA] * 3
        ),
    )
    return f(ids, hash_rows[0], hash_rows[1], hash_rows[2],
             imp_cols[0], imp_cols[1], imp_cols[2], emb)


def kernel(X, hash_vals, word_importance, embedding_matrix):
    ids = X.reshape(_N)
    hash_rows = [hash_vals[h] for h in range(_NUM_HASH)]
    imp_cols = [word_importance[:, h] for h in range(_NUM_HASH)]
    out = _sc_lookup(ids, hash_rows, imp_cols, embedding_matrix)
    return out.reshape(X.shape[0], X.shape[1], _EMB)


# speedup vs baseline: 1.0713x; 1.0713x over previous
import jax
import jax.numpy as jnp
from jax import lax
from jax.experimental import pallas as pl
from jax.experimental.pallas import tpu as pltpu
from jax.experimental.pallas import tpu_sc as plsc

_NUM_HASH = 3
_NUM_WORDS = 1000000
_EMB = 64
_NC = 2
_NS = 16
_NW = _NC * _NS
_LANES = 16

_N = 4096 * 50
_NPW = _N // _NW
_C = 160
_NCHUNK = _NPW // _C
_NPAIR = _NCHUNK // 2


def _body(ids_hbm, hash_hbm, imp0_hbm, imp1_hbm, imp2_hbm, emb_hbm, out_hbm,
          ids_v,
          bidxa, bidx1a, bidx2a, iidxa, buck0a, buck1a, buck2a,
          wimp0a, wimp1a, wimp2a, emb0a, emb1a, emb2a, outba,
          bidxb, bidx1b, bidx2b, iidxb, buck0b, buck1b, buck2b,
          wimp0b, wimp1b, wimp2b, emb0b, emb1b, emb2b, outbb,
          sem_s, sem_e, sem_o):
    sets = (
        dict(bidx=bidxa, bidx1=bidx1a, bidx2=bidx2a, iidx=iidxa,
             buck=(buck0a, buck1a, buck2a), imp=(wimp0a, wimp1a, wimp2a),
             emb=(emb0a, emb1a, emb2a), outb=outba),
        dict(bidx=bidxb, bidx1=bidx1b, bidx2=bidx2b, iidx=iidxb,
             buck=(buck0b, buck1b, buck2b), imp=(wimp0b, wimp1b, wimp2b),
             emb=(emb0b, emb1b, emb2b), outb=outbb),
    )
    imp_hbms = (imp0_hbm, imp1_hbm, imp2_hbm)
    wid = lax.axis_index("s") * _NC + lax.axis_index("c")
    base = wid * _NPW
    pltpu.sync_copy(ids_hbm.at[pl.ds(base, _NPW)], ids_v)

    def build_idx(off, s):
        for j in range(_C // _LANES):
            dsl = pl.ds(j * _LANES, _LANES)
            v = ids_v[pl.ds(off + j * _LANES, _LANES)]
            s['bidx'][dsl] = v
            s['bidx1'][dsl] = v + _NUM_WORDS
            s['bidx2'][dsl] = v + 2 * _NUM_WORDS
            vi = v + 3
            s['iidx'][dsl] = jnp.where(vi >= _NUM_WORDS, vi - _NUM_WORDS, vi)

    def fire_w1(s):
        bs = (s['bidx'], s['bidx1'], s['bidx2'])
        for h in range(_NUM_HASH):
            pltpu.async_copy(hash_hbm.at[bs[h]], s['buck'][h], sem_s)
            pltpu.async_copy(imp_hbms[h].at[s['iidx']], s['imp'][h], sem_s)

    def wait_w1(s):
        bs = (s['bidx'], s['bidx1'], s['bidx2'])
        for h in range(_NUM_HASH):
            pltpu.make_async_copy(hash_hbm.at[bs[h]], s['buck'][h],
                                  sem_s).wait()
            pltpu.make_async_copy(imp_hbms[h].at[s['iidx']], s['imp'][h],
                                  sem_s).wait()

    def fire_w2(s):
        for h in range(_NUM_HASH):
            pltpu.async_copy(emb_hbm.at[s['buck'][h]], s['emb'][h], sem_e)

    def wait_w2(s):
        for h in range(_NUM_HASH):
            pltpu.make_async_copy(emb_hbm.at[s['buck'][h]], s['emb'][h],
                                  sem_e).wait()

    def combine(s):
        e0, e1, e2 = s['emb']
        i0, i1, i2 = s['imp']
        ob = s['outb']
        dnums = lax.GatherDimensionNumbers(
            offset_dims=(), collapsed_slice_dims=(0,), start_index_map=(0,))

        def bcast(vec, k):
            idx = jnp.full((_LANES, 1), k, jnp.int32)
            return lax.gather(vec, idx, dnums, (1,),
                              mode=lax.GatherScatterMode.PROMISE_IN_BOUNDS)

        def grp(g, c):
            t0 = g * _LANES
            m0 = i0[pl.ds(t0, _LANES)]
            m1 = i1[pl.ds(t0, _LANES)]
            m2 = i2[pl.ds(t0, _LANES)]
            for k in range(_LANES):
                t = t0 + k
                w0 = bcast(m0, k)
                w1 = bcast(m1, k)
                w2 = bcast(m2, k)
                for d in range(_EMB // _LANES):
                    esl = pl.ds(d * _LANES, _LANES)
                    ob[t, esl] = (e0[t, esl] * w0 + e1[t, esl] * w1
                                  + e2[t, esl] * w2)
            return c
        lax.fori_loop(0, _C // _LANES, grp, 0)

    def chunk_step(i, b):
        s = sets[b]
        nxt = sets[1 - b]
        j = 2 * i + b
        off = j * _C

        @pl.when(j >= 2)
        def _():
            pltpu.make_async_copy(
                s['outb'], out_hbm.at[pl.ds(base + off - 2 * _C, _C)],
                sem_o).wait()

        wait_w2(s)

        @pl.when(j + 2 < _NCHUNK)
        def _():
            build_idx(off + 2 * _C, s)

        @pl.when(j + 1 < _NCHUNK)
        def _():
            wait_w1(nxt)
            fire_w2(nxt)

        combine(s)
        pltpu.async_copy(s['outb'], out_hbm.at[pl.ds(base + off, _C)], sem_o)

        @pl.when(j + 2 < _NCHUNK)
        def _():
            fire_w1(s)

    def pair(i, carry):
        chunk_step(i, 0)
        chunk_step(i, 1)
        return carry

    build_idx(0, sets[0])
    fire_w1(sets[0])
    build_idx(_C, sets[1])
    fire_w1(sets[1])
    wait_w1(sets[0])
    fire_w2(sets[0])
    lax.fori_loop(0, _NPAIR, pair, 0)
    pltpu.make_async_copy(
        sets[0]['outb'], out_hbm.at[pl.ds(base + _NPW - 2 * _C, _C)],
        sem_o).wait()
    pltpu.make_async_copy(
        sets[1]['outb'], out_hbm.at[pl.ds(base + _NPW - _C, _C)],
        sem_o).wait()


def _sc_lookup(ids, hash_flat, imp_cols, emb):
    mesh = plsc.VectorSubcoreMesh(core_axis_name="c", subcore_axis_name="s")
    one_set = (
        [pltpu.VMEM((_C,), jnp.int32)] * 4
        + [pltpu.VMEM((_C,), jnp.int32)] * 3
        + [pltpu.VMEM((_C,), jnp.float32)] * 3
        + [pltpu.VMEM((_C, _EMB), jnp.float32)] * 3
        + [pltpu.VMEM((_C, _EMB), jnp.float32)]
    )
    f = pl.kernel(
        _body,
        out_type=jax.ShapeDtypeStruct((_N, _EMB), jnp.float32),
        mesh=mesh,
        compiler_params=pltpu.CompilerParams(needs_layout_passes=False,
                                             use_tc_tiling_on_sc=False),
        scratch_types=(
            [pltpu.VMEM((_NPW,), jnp.int32)]
            + one_set + one_set
            + [pltpu.SemaphoreType.DMA] * 3
        ),
    )
    return f(ids, hash_flat, imp_cols[0], imp_cols[1], imp_cols[2], emb)


def kernel(X, hash_vals, word_importance, embedding_matrix):
    ids = X.reshape(_N)
    hash_flat = hash_vals.reshape(_NUM_HASH * _NUM_WORDS)
    imp_cols = [word_importance[:, h] for h in range(_NUM_HASH)]
    out = _sc_lookup(ids, hash_flat, imp_cols, embedding_matrix)
    return out.reshape(X.shape[0], X.shape[1], _EMB)
